# baseline (speedup 1.0000x reference)
import functools

import jax
import jax.numpy as jnp
from jax.experimental import pallas as pl
from jax.experimental.pallas import tpu as pltpu


def _chamfer_body(x1_ref, x2_ref, out_ref, d2min_ref, *, ni_blocks, inv_n):
    i = pl.program_id(1)

    x1 = x1_ref[0]

    t = jax.lax.dot_general(
        x1 * -2.0, x2_ref[0], (((0,), (0,)), ((), ())),
        preferred_element_type=jnp.float32,
    )
    sq1 = jnp.sum(x1 * x1, axis=0, keepdims=True)
    sq2 = jnp.sum(x2_ref[0] * x2_ref[0], axis=0, keepdims=True)

    row_min = jnp.min(t + sq2, axis=1, keepdims=True) + sq1.T
    s1_sum = jnp.sum(row_min)

    col_min = jnp.min(t + sq1.T, axis=0, keepdims=True)

    @pl.when(i == 0)
    def _init():
        d2min_ref[...] = col_min
        out_ref[0, 0, 0] = 0.0

    @pl.when(i > 0)
    def _acc():
        d2min_ref[...] = jnp.minimum(d2min_ref[...], col_min)

    out_ref[0, 0, 0] += s1_sum * inv_n

    @pl.when(i == ni_blocks - 1)
    def _flush():
        out_ref[0, 0, 0] += jnp.sum(d2min_ref[...] + sq2) * inv_n


def kernel(xyz1, xyz2):
    B, N1, _ = xyz1.shape
    _, N2, _ = xyz2.shape
    BI = 1024
    ni_blocks = N1 // BI

    x1t = jnp.transpose(xyz1, (0, 2, 1))
    x2t = jnp.transpose(xyz2, (0, 2, 1))

    body = functools.partial(
        _chamfer_body, ni_blocks=ni_blocks, inv_n=1.0 / float(B * N1)
    )

    partial = pl.pallas_call(
        body,
        grid=(B, ni_blocks),
        in_specs=[
            pl.BlockSpec((1, 3, BI), lambda b, i: (b, 0, i)),
            pl.BlockSpec((1, 3, N2), lambda b, i: (b, 0, 0)),
        ],
        out_specs=pl.BlockSpec((1, 1, 1), lambda b, i: (b, 0, 0), memory_space=pltpu.SMEM),
        out_shape=jax.ShapeDtypeStruct((B, 1, 1), jnp.float32),
        scratch_shapes=[pltpu.VMEM((1, N2), jnp.float32)],
        compiler_params=pltpu.CompilerParams(
            dimension_semantics=("parallel", "arbitrary"),
        ),
    )(x1t, x2t)
    return jnp.sum(partial)


# speedup vs baseline: 1.1045x; 1.1045x over previous
import functools

import jax
import jax.numpy as jnp
from jax.experimental import pallas as pl
from jax.experimental.pallas import tpu as pltpu


def _chamfer_body(x1_ref, x2_ref, out_ref, d2min_ref, *, ni_blocks, inv_n):
    i = pl.program_id(1)

    x1 = x1_ref[0]

    t = jax.lax.dot_general(
        x1 * -2.0, x2_ref[0], (((0,), (0,)), ((), ())),
        preferred_element_type=jnp.float32,
    )
    sq1 = jnp.sum(x1 * x1, axis=0, keepdims=True)
    sq2 = jnp.sum(x2_ref[0] * x2_ref[0], axis=0, keepdims=True)

    row_min = jnp.min(t + sq2, axis=1, keepdims=True) + sq1.T
    s1_sum = jnp.sum(row_min)

    col_min = jnp.min(t + sq1.T, axis=0, keepdims=True)

    @pl.when(i == 0)
    def _init():
        d2min_ref[...] = col_min
        out_ref[0, 0, 0] = 0.0

    @pl.when(i > 0)
    def _acc():
        d2min_ref[...] = jnp.minimum(d2min_ref[...], col_min)

    out_ref[0, 0, 0] += s1_sum * inv_n

    @pl.when(i == ni_blocks - 1)
    def _flush():
        out_ref[0, 0, 0] += jnp.sum(d2min_ref[...] + sq2) * inv_n


def kernel(xyz1, xyz2):
    B, N1, _ = xyz1.shape
    _, N2, _ = xyz2.shape
    BI = 2048
    ni_blocks = N1 // BI

    x1t = jnp.transpose(xyz1, (0, 2, 1))
    x2t = jnp.transpose(xyz2, (0, 2, 1))

    body = functools.partial(
        _chamfer_body, ni_blocks=ni_blocks, inv_n=1.0 / float(B * N1)
    )

    partial = pl.pallas_call(
        body,
        grid=(B, ni_blocks),
        in_specs=[
            pl.BlockSpec((1, 3, BI), lambda b, i: (b, 0, i)),
            pl.BlockSpec((1, 3, N2), lambda b, i: (b, 0, 0)),
        ],
        out_specs=pl.BlockSpec((1, 1, 1), lambda b, i: (b, 0, 0), memory_space=pltpu.SMEM),
        out_shape=jax.ShapeDtypeStruct((B, 1, 1), jnp.float32),
        scratch_shapes=[pltpu.VMEM((1, N2), jnp.float32)],
        compiler_params=pltpu.CompilerParams(
            dimension_semantics=("parallel", "arbitrary"),
        ),
    )(x1t, x2t)
    return jnp.sum(partial)


# speedup vs baseline: 1.1903x; 1.0777x over previous
import functools

import jax
import jax.numpy as jnp
from jax.experimental import pallas as pl
from jax.experimental.pallas import tpu as pltpu


def _chamfer_body(x1_ref, x2_ref, out_ref, d2min_ref, *, ni_blocks, inv_n):
    i = pl.program_id(1)

    x1 = x1_ref[0]

    t = jax.lax.dot_general(
        x1 * -2.0, x2_ref[0], (((0,), (0,)), ((), ())),
        preferred_element_type=jnp.float32,
    )
    sq1 = jnp.sum(x1 * x1, axis=0, keepdims=True)
    sq2 = jnp.sum(x2_ref[0] * x2_ref[0], axis=0, keepdims=True)

    row_min = jnp.min(t + sq2, axis=1, keepdims=True) + sq1.T
    s1_sum = jnp.sum(row_min)

    col_min = jnp.min(t + sq1.T, axis=0, keepdims=True)

    @pl.when(i == 0)
    def _init():
        d2min_ref[...] = col_min
        out_ref[0, 0, 0] = 0.0

    @pl.when(i > 0)
    def _acc():
        d2min_ref[...] = jnp.minimum(d2min_ref[...], col_min)

    out_ref[0, 0, 0] += s1_sum * inv_n

    @pl.when(i == ni_blocks - 1)
    def _flush():
        out_ref[0, 0, 0] += jnp.sum(d2min_ref[...] + sq2) * inv_n


def kernel(xyz1, xyz2):
    B, N1, _ = xyz1.shape
    _, N2, _ = xyz2.shape
    BI = 4096
    ni_blocks = N1 // BI

    x1t = jnp.transpose(xyz1, (0, 2, 1))
    x2t = jnp.transpose(xyz2, (0, 2, 1))

    body = functools.partial(
        _chamfer_body, ni_blocks=ni_blocks, inv_n=1.0 / float(B * N1)
    )

    partial = pl.pallas_call(
        body,
        grid=(B, ni_blocks),
        in_specs=[
            pl.BlockSpec((1, 3, BI), lambda b, i: (b, 0, i)),
            pl.BlockSpec((1, 3, N2), lambda b, i: (b, 0, 0)),
        ],
        out_specs=pl.BlockSpec((1, 1, 1), lambda b, i: (b, 0, 0), memory_space=pltpu.SMEM),
        out_shape=jax.ShapeDtypeStruct((B, 1, 1), jnp.float32),
        scratch_shapes=[pltpu.VMEM((1, N2), jnp.float32)],
        compiler_params=pltpu.CompilerParams(
            dimension_semantics=("parallel", "arbitrary"),
        ),
    )(x1t, x2t)
    return jnp.sum(partial)


# speedup vs baseline: 1.2009x; 1.0089x over previous
import functools

import jax
import jax.numpy as jnp
from jax.experimental import pallas as pl
from jax.experimental.pallas import tpu as pltpu


def _chamfer_body(x1_ref, x2_ref, out_ref, *, inv_n):
    x1 = x1_ref[0]
    x2 = x2_ref[0]

    t = jax.lax.dot_general(
        x1 * -2.0, x2, (((0,), (0,)), ((), ())),
        preferred_element_type=jnp.float32,
    )
    sq1 = jnp.sum(x1 * x1, axis=0, keepdims=True)
    sq2 = jnp.sum(x2 * x2, axis=0, keepdims=True)

    row_min = jnp.min(t + sq2, axis=1, keepdims=True)
    col_min = jnp.min(t + sq1.T, axis=0, keepdims=True)

    total = jnp.sum(row_min) + jnp.sum(col_min) + jnp.sum(sq1) + jnp.sum(sq2)
    out_ref[0, 0, 0] = total * inv_n


def kernel(xyz1, xyz2):
    B, N1, _ = xyz1.shape
    _, N2, _ = xyz2.shape

    x1t = jnp.transpose(xyz1, (0, 2, 1))
    x2t = jnp.transpose(xyz2, (0, 2, 1))

    body = functools.partial(_chamfer_body, inv_n=1.0 / float(B * N1))

    partial = pl.pallas_call(
        body,
        grid=(B,),
        in_specs=[
            pl.BlockSpec((1, 3, N1), lambda b: (b, 0, 0)),
            pl.BlockSpec((1, 3, N2), lambda b: (b, 0, 0)),
        ],
        out_specs=pl.BlockSpec(
            (1, 1, 1), lambda b: (b, 0, 0), memory_space=pltpu.SMEM
        ),
        out_shape=jax.ShapeDtypeStruct((B, 1, 1), jnp.float32),
        compiler_params=pltpu.CompilerParams(
            dimension_semantics=("parallel",),
        ),
    )(x1t, x2t)
    return jnp.sum(partial)
